# baseline (speedup 1.0000x reference)
import jax
import jax.numpy as jnp
from jax.experimental import pallas as pl
from jax.experimental.pallas import tpu as pltpu

BLOCK_ROWS = 512


def _make_kl_sum_kernel(grid, inv_n):
    def _kl_sum_kernel(x_ref, t_ref, o_ref):
        i = pl.program_id(0)
        t = t_ref[...]
        x = x_ref[...]
        safe_t = jnp.where(t > 0, t, 1.0)
        kl = t * jnp.log(safe_t) - t * x
        s = jnp.sum(kl)

        @pl.when(i == 0)
        def _init():
            o_ref[0, 0] = 0.0

        o_ref[0, 0] += s

        @pl.when(i == grid - 1)
        def _finalize():
            o_ref[0, 0] *= inv_n

    return _kl_sum_kernel


def kernel(inputs, targets):
    rows, cols = inputs.shape
    grid = rows // BLOCK_ROWS

    out = pl.pallas_call(
        _make_kl_sum_kernel(grid, 1.0 / (rows * cols)),
        grid=(grid,),
        in_specs=[
            pl.BlockSpec((BLOCK_ROWS, cols), lambda i: (i, 0)),
            pl.BlockSpec((BLOCK_ROWS, cols), lambda i: (i, 0)),
        ],
        out_specs=pl.BlockSpec((1, 1), lambda i: (0, 0), memory_space=pltpu.SMEM),
        out_shape=jax.ShapeDtypeStruct((1, 1), jnp.float32),
        compiler_params=pltpu.CompilerParams(
            dimension_semantics=("arbitrary",),
        ),
    )(inputs, targets)
    return out.reshape(())


# speedup vs baseline: 1.0064x; 1.0064x over previous
import jax
import jax.numpy as jnp
from jax.experimental import pallas as pl
from jax.experimental.pallas import tpu as pltpu

BLOCK_ROWS = 512


def _make_kl_sum_kernel(grid, inv_n):
    def _kl_sum_kernel(x_ref, t_ref, o_ref):
        i = pl.program_id(0)
        t = t_ref[...]
        x = x_ref[...]
        safe_t = jnp.maximum(t, jnp.float32(1.1754944e-38))
        kl = t * (jnp.log(safe_t) - x)
        s = jnp.sum(kl)

        @pl.when(i == 0)
        def _init():
            o_ref[0, 0] = 0.0

        o_ref[0, 0] += s

        @pl.when(i == grid - 1)
        def _finalize():
            o_ref[0, 0] *= inv_n

    return _kl_sum_kernel


def kernel(inputs, targets):
    rows, cols = inputs.shape
    grid = rows // BLOCK_ROWS

    out = pl.pallas_call(
        _make_kl_sum_kernel(grid, 1.0 / (rows * cols)),
        grid=(grid,),
        in_specs=[
            pl.BlockSpec((BLOCK_ROWS, cols), lambda i: (i, 0)),
            pl.BlockSpec((BLOCK_ROWS, cols), lambda i: (i, 0)),
        ],
        out_specs=pl.BlockSpec((1, 1), lambda i: (0, 0), memory_space=pltpu.SMEM),
        out_shape=jax.ShapeDtypeStruct((1, 1), jnp.float32),
        compiler_params=pltpu.CompilerParams(
            dimension_semantics=("arbitrary",),
        ),
    )(inputs, targets)
    return out.reshape(())
